# baseline (speedup 1.0000x reference)
import functools

import jax
import jax.numpy as jnp
import numpy as np
from jax import lax
from jax.experimental import pallas as pl
from jax.experimental.pallas import tpu as pltpu
from jax.experimental.pallas import tpu_sc as plsc

N = 10000
E = 320000
D = 128
DW = D // 2
K = 32.0
EPS = 1e-5

NC = 2
NS = 16
NW = NC * NS
EPW = E // NW
CB = 80
NCHUNK = EPW // CB


def _silu(v):
    return v * jax.nn.sigmoid(v)


def _pre_body(h_ref, w1a_ref, w1b_ref, b1_ref, hp_ref, hq_ref):
    h = h_ref[...]
    hp_ref[...] = jnp.dot(h, w1a_ref[...], preferred_element_type=jnp.float32) + b1_ref[...]
    hq_ref[...] = jnp.dot(h, w1b_ref[...], preferred_element_type=jnp.float32)


def _node_pre(h, w1a, w1b, b1):
    blk = 2000
    return pl.pallas_call(
        _pre_body,
        grid=(N // blk,),
        in_specs=[
            pl.BlockSpec((blk, D), lambda i: (i, 0)),
            pl.BlockSpec((D, D), lambda i: (0, 0)),
            pl.BlockSpec((D, D), lambda i: (0, 0)),
            pl.BlockSpec((1, D), lambda i: (0, 0)),
        ],
        out_specs=[
            pl.BlockSpec((blk, D), lambda i: (i, 0)),
            pl.BlockSpec((blk, D), lambda i: (i, 0)),
        ],
        out_shape=[
            jax.ShapeDtypeStruct((N, D), jnp.float32),
            jax.ShapeDtypeStruct((N, D), jnp.float32),
        ],
    )(h, w1a, w1b, b1)


def _edge_gather_body(hp_hbm, hq_hbm, xx_hbm, xy_hbm, xz_hbm, row_hbm, col_hbm,
                      t_hbm, d_hbm,
                      rows_v, cols_v,
                      gp0, gp1, gq0, gq1, t0, t1, dv0, dv1,
                      xr0a, xr1a, xr2a, xc0a, xc1a, xc2a,
                      xr0b, xr1b, xr2b, xc0b, xc1b, xc2b,
                      sp0, sp1, sq0, sq1, sx0, sx1, st0, st1, sd0, sd1):
    wid = lax.axis_index("s") * NC + lax.axis_index("c")
    ebase = wid * EPW
    pltpu.sync_copy(row_hbm.at[pl.ds(ebase, EPW)], rows_v)
    pltpu.sync_copy(col_hbm.at[pl.ds(ebase, EPW)], cols_v)

    GP = [gp0, gp1]
    GQ = [gq0, gq1]
    TV = [t0, t1]
    DV = [dv0, dv1]
    XR = [[xr0a, xr1a, xr2a], [xr0b, xr1b, xr2b]]
    XC = [[xc0a, xc1a, xc2a], [xc0b, xc1b, xc2b]]
    SP = [sp0, sp1]
    SQ = [sq0, sq1]
    SX = [sx0, sx1]
    ST = [st0, st1]
    SD = [sd0, sd1]
    XH = [xx_hbm, xy_hbm, xz_hbm]

    def issue(k, b):
        sl = pl.ds(k * CB, CB)
        pltpu.async_copy(hp_hbm.at[rows_v.at[sl]], GP[b], SP[b])
        pltpu.async_copy(hq_hbm.at[cols_v.at[sl]], GQ[b], SQ[b])
        for j in range(3):
            pltpu.async_copy(XH[j].at[rows_v.at[sl]], XR[b][j], SX[b])
            pltpu.async_copy(XH[j].at[cols_v.at[sl]], XC[b][j], SX[b])

    def wait_gathers(b):
        sl = pl.ds(0, CB)
        pltpu.make_async_copy(hp_hbm.at[rows_v.at[sl]], GP[b], SP[b]).wait()
        pltpu.make_async_copy(hq_hbm.at[cols_v.at[sl]], GQ[b], SQ[b]).wait()
        for j in range(3):
            pltpu.make_async_copy(XH[j].at[rows_v.at[sl]], XR[b][j], SX[b]).wait()
            pltpu.make_async_copy(XH[j].at[cols_v.at[sl]], XC[b][j], SX[b]).wait()

    def wait_store(b):
        pltpu.make_async_copy(TV[b], t_hbm.at[pl.ds(ebase, CB)], ST[b]).wait()
        pltpu.make_async_copy(DV[b], d_hbm.at[pl.ds(ebase, CB)], SD[b]).wait()

    def compute(k, b):
        def dij_body(g, c2):
            sl = pl.ds(g * 16, 16)
            d0 = XR[b][0][sl] - XC[b][0][sl]
            d1 = XR[b][1][sl] - XC[b][1][sl]
            d2 = XR[b][2][sl] - XC[b][2][sl]
            DV[b][sl] = d0 * d0 + d1 * d1 + d2 * d2
            return c2

        lax.fori_loop(0, CB // 16, dij_body, 0, unroll=True)

        def edge_body(e, c2):
            for c in range(D // 16):
                sl = pl.ds(c * 16, 16)
                TV[b][e, sl] = GP[b][e, sl] + GQ[b][e, sl]
            return c2

        lax.fori_loop(0, CB, edge_body, 0, unroll=2)
        pltpu.async_copy(TV[b], t_hbm.at[pl.ds(ebase + k * CB, CB)], ST[b])
        pltpu.async_copy(DV[b], d_hbm.at[pl.ds(ebase + k * CB, CB)], SD[b])

    issue(0, 0)

    def pair_body(j, carry):
        for ph in range(2):
            k = 2 * j + ph
            b = ph

            @pl.when(k + 1 < NCHUNK)
            def _():
                issue(k + 1, 1 - ph)

            @pl.when(k < NCHUNK)
            def _():
                wait_gathers(b)

            @pl.when(jnp.logical_and(k >= 2, k < NCHUNK))
            def _():
                wait_store(b)

            @pl.when(k < NCHUNK)
            def _():
                compute(k, b)

        return carry

    lax.fori_loop(0, (NCHUNK + 2) // 2, pair_body, 0)
    wait_store(0)
    wait_store(1)


def _edge_gather(hp, hq, xx, xy, xz, row, col):
    mesh = plsc.VectorSubcoreMesh(core_axis_name="c", subcore_axis_name="s")
    kern = functools.partial(
        pl.kernel,
        mesh=mesh,
        out_type=(
            jax.ShapeDtypeStruct((E, D), jnp.float32),
            jax.ShapeDtypeStruct((E,), jnp.float32),
        ),
        scratch_types=(
            [
                pltpu.VMEM((EPW,), jnp.int32),
                pltpu.VMEM((EPW,), jnp.int32),
            ]
            + [pltpu.VMEM((CB, D), jnp.float32)] * 6
            + [pltpu.VMEM((CB,), jnp.float32)] * 2
            + [pltpu.VMEM((CB,), jnp.float32)] * 12
            + [pltpu.SemaphoreType.DMA] * 10
        ),
    )(_edge_gather_body)
    return kern(hp, hq, xx, xy, xz, row, col)


def _mlp_body(t_ref, d_ref, w1c_ref, w2_ref, b2_ref, m_ref):
    a = _silu(t_ref[...] + d_ref[...] * w1c_ref[...])
    z = jnp.dot(a, w2_ref[...], preferred_element_type=jnp.float32) + b2_ref[...]
    m_ref[...] = _silu(z)


def _edge_mlp(t, dij, w1c, w2, b2):
    blk = 1000
    return pl.pallas_call(
        _mlp_body,
        grid=(E // blk,),
        in_specs=[
            pl.BlockSpec((blk, D), lambda i: (i, 0)),
            pl.BlockSpec((blk, 1), lambda i: (i, 0)),
            pl.BlockSpec((1, D), lambda i: (0, 0)),
            pl.BlockSpec((D, D), lambda i: (0, 0)),
            pl.BlockSpec((1, D), lambda i: (0, 0)),
        ],
        out_specs=pl.BlockSpec((blk, D), lambda i: (i, 0)),
        out_shape=jax.ShapeDtypeStruct((E, D), jnp.float32),
    )(t, dij, w1c, w2, b2)


WB = 624
WREM = N - NS * WB


def _scatter_body(m_hbm, row_hbm, zeros_hbm, out_hbm,
                  row0, row1, m0, m1, agg_sh, sf0, sf1):
    c = lax.axis_index("c")
    s = lax.axis_index("s")
    wid = s * NC + c
    ebase = wid * EPW
    RV = [row0, row1]
    MV = [m0, m1]
    SF = [sf0, sf1]

    def fetch(k, b):
        base = ebase + k * CB
        pltpu.async_copy(row_hbm.at[pl.ds(base, CB)], RV[b], SF[b])
        pltpu.async_copy(m_hbm.at[pl.ds(base, CB)], MV[b], SF[b])

    def wait_fetch(b):
        pltpu.make_async_copy(row_hbm.at[pl.ds(ebase, CB)], RV[b], SF[b]).wait()
        pltpu.make_async_copy(m_hbm.at[pl.ds(ebase, CB)], MV[b], SF[b]).wait()

    fetch(0, 0)
    pltpu.sync_copy(zeros_hbm, agg_sh.at[pl.ds(s * WB, WB)])

    @pl.when(s == 0)
    def _():
        pltpu.sync_copy(zeros_hbm.at[pl.ds(0, WREM)], agg_sh.at[pl.ds(NS * WB, WREM)])

    plsc.subcore_barrier()

    def pair_body(j, carry):
        for ph in range(2):
            k = 2 * j + ph
            b = ph

            @pl.when(k + 1 < NCHUNK)
            def _():
                fetch(k + 1, 1 - ph)

            @pl.when(k < NCHUNK)
            def _():
                wait_fetch(b)
                pltpu.sync_copy(MV[b], agg_sh.at[RV[b]], add=True)

        return carry

    lax.fori_loop(0, (NCHUNK + 2) // 2, pair_body, 0)
    plsc.subcore_barrier()
    pltpu.sync_copy(agg_sh.at[pl.ds(s * WB, WB)], out_hbm.at[c, pl.ds(s * WB, WB)])

    @pl.when(s == 0)
    def _():
        pltpu.sync_copy(agg_sh.at[pl.ds(NS * WB, WREM)],
                        out_hbm.at[c, pl.ds(NS * WB, WREM)])


def _scatter_add(m, row, zeros):
    mesh = plsc.VectorSubcoreMesh(core_axis_name="c", subcore_axis_name="s")
    kern = functools.partial(
        pl.kernel,
        mesh=mesh,
        out_type=jax.ShapeDtypeStruct((NC, N, D), jnp.float32),
        scratch_types=[
            pltpu.VMEM((CB,), jnp.int32),
            pltpu.VMEM((CB,), jnp.int32),
            pltpu.VMEM((CB, D), jnp.float32),
            pltpu.VMEM((CB, D), jnp.float32),
            pltpu.VMEM_SHARED((N, D), jnp.float32),
            pltpu.SemaphoreType.DMA,
            pltpu.SemaphoreType.DMA,
        ],
    )(_scatter_body)
    return kern(m, row, zeros)


def _node_body(h_ref, p_ref, w3h_ref, w3a_ref, b3_ref, w4_ref, b4_ref,
               gamma_ref, beta_ref, o_ref):
    h = h_ref[...]
    p = p_ref[...]
    agg = (p[0] + p[1]) * (1.0 / K)
    z = (jnp.dot(h, w3h_ref[...], preferred_element_type=jnp.float32)
         + jnp.dot(agg, w3a_ref[...], preferred_element_type=jnp.float32)
         + b3_ref[...])
    u = jnp.dot(_silu(z), w4_ref[...], preferred_element_type=jnp.float32) + b4_ref[...]
    y = h + u
    mu = jnp.mean(y, axis=-1, keepdims=True)
    var = jnp.mean((y - mu) ** 2, axis=-1, keepdims=True)
    o_ref[...] = (y - mu) / jnp.sqrt(var + EPS) * gamma_ref[...] + beta_ref[...]


def _node_mlp(h, partials, w3h, w3a, b3, w4, b4, gamma, beta):
    blk = 2000
    return pl.pallas_call(
        _node_body,
        grid=(N // blk,),
        in_specs=[
            pl.BlockSpec((blk, D), lambda i: (i, 0)),
            pl.BlockSpec((NC, blk, D), lambda i: (0, i, 0)),
            pl.BlockSpec((D, D), lambda i: (0, 0)),
            pl.BlockSpec((D, D), lambda i: (0, 0)),
            pl.BlockSpec((1, D), lambda i: (0, 0)),
            pl.BlockSpec((D, D), lambda i: (0, 0)),
            pl.BlockSpec((1, D), lambda i: (0, 0)),
            pl.BlockSpec((1, D), lambda i: (0, 0)),
            pl.BlockSpec((1, D), lambda i: (0, 0)),
        ],
        out_specs=pl.BlockSpec((blk, D), lambda i: (i, 0)),
        out_shape=jax.ShapeDtypeStruct((N, D), jnp.float32),
    )(h, partials, w3h, w3a, b3, w4, b4, gamma, beta)


def kernel(h, x, edge_index, W1, b1, W2, b2, W3, b3, W4, b4, gamma, beta):
    row = edge_index[0].astype(jnp.int32)
    col = edge_index[1].astype(jnp.int32)
    xf = x.astype(jnp.float32)
    w1a = W1[:D]
    w1b = W1[D:2 * D]
    w1c = W1[2 * D]
    hp, hq = _node_pre(h, w1a, w1b, b1.reshape(1, D))
    t, dij = _edge_gather(hp, hq, xf[:, 0], xf[:, 1], xf[:, 2], row, col)
    m = _edge_mlp(t, dij.reshape(E, 1), w1c.reshape(1, D), W2, b2.reshape(1, D))
    zeros = jnp.zeros((WB, D), jnp.float32)
    partials = _scatter_add(m, row, zeros)
    return _node_mlp(h, partials, W3[:D], W3[D:], b3.reshape(1, D), W4,
                     b4.reshape(1, D), gamma.reshape(1, D), beta.reshape(1, D))


# speedup vs baseline: 1.0703x; 1.0703x over previous
import functools

import jax
import jax.numpy as jnp
import numpy as np
from jax import lax
from jax.experimental import pallas as pl
from jax.experimental.pallas import tpu as pltpu
from jax.experimental.pallas import tpu_sc as plsc

N = 10000
E = 320000
D = 128
DW = D // 2
K = 32.0
EPS = 1e-5

NC = 2
NS = 16
NW = NC * NS
EPW = E // NW
CB = 80
NCHUNK = EPW // CB
ESPLIT = 163840
assert ESPLIT % (NW * CB) == 0 and (E - ESPLIT) % (NW * CB) == 0


def _silu(v):
    return v * jax.nn.sigmoid(v)


def _pre_body(h_ref, w1a_ref, w1b_ref, b1_ref, hp_ref, hq_ref):
    h = h_ref[...]
    hp_ref[...] = jnp.dot(h, w1a_ref[...], preferred_element_type=jnp.float32) + b1_ref[...]
    hq_ref[...] = jnp.dot(h, w1b_ref[...], preferred_element_type=jnp.float32)


def _node_pre(h, w1a, w1b, b1):
    blk = 2000
    return pl.pallas_call(
        _pre_body,
        grid=(N // blk,),
        in_specs=[
            pl.BlockSpec((blk, D), lambda i: (i, 0)),
            pl.BlockSpec((D, D), lambda i: (0, 0)),
            pl.BlockSpec((D, D), lambda i: (0, 0)),
            pl.BlockSpec((1, D), lambda i: (0, 0)),
        ],
        out_specs=[
            pl.BlockSpec((blk, D), lambda i: (i, 0)),
            pl.BlockSpec((blk, D), lambda i: (i, 0)),
        ],
        out_shape=[
            jax.ShapeDtypeStruct((N, D), jnp.float32),
            jax.ShapeDtypeStruct((N, D), jnp.float32),
        ],
    )(h, w1a, w1b, b1)


def _edge_gather_body(e_off, epw, nchunk,
                      hp_hbm, hq_hbm, xx_hbm, xy_hbm, xz_hbm, row_hbm, col_hbm,
                      t_hbm, d_hbm,
                      rows_v, cols_v,
                      gp0, gp1, gq0, gq1, t0, t1, dv0, dv1,
                      xr0a, xr1a, xr2a, xc0a, xc1a, xc2a,
                      xr0b, xr1b, xr2b, xc0b, xc1b, xc2b,
                      sp0, sp1, sq0, sq1, sx0, sx1, st0, st1, sd0, sd1):
    wid = lax.axis_index("s") * NC + lax.axis_index("c")
    ebase = wid * epw
    pltpu.sync_copy(row_hbm.at[pl.ds(e_off + ebase, epw)], rows_v)
    pltpu.sync_copy(col_hbm.at[pl.ds(e_off + ebase, epw)], cols_v)

    GP = [gp0, gp1]
    GQ = [gq0, gq1]
    TV = [t0, t1]
    DV = [dv0, dv1]
    XR = [[xr0a, xr1a, xr2a], [xr0b, xr1b, xr2b]]
    XC = [[xc0a, xc1a, xc2a], [xc0b, xc1b, xc2b]]
    SP = [sp0, sp1]
    SQ = [sq0, sq1]
    SX = [sx0, sx1]
    ST = [st0, st1]
    SD = [sd0, sd1]
    XH = [xx_hbm, xy_hbm, xz_hbm]

    def issue(k, b):
        sl = pl.ds(k * CB, CB)
        pltpu.async_copy(hp_hbm.at[rows_v.at[sl]], GP[b], SP[b])
        pltpu.async_copy(hq_hbm.at[cols_v.at[sl]], GQ[b], SQ[b])
        for j in range(3):
            pltpu.async_copy(XH[j].at[rows_v.at[sl]], XR[b][j], SX[b])
            pltpu.async_copy(XH[j].at[cols_v.at[sl]], XC[b][j], SX[b])

    def wait_gathers(b):
        sl = pl.ds(0, CB)
        pltpu.make_async_copy(hp_hbm.at[rows_v.at[sl]], GP[b], SP[b]).wait()
        pltpu.make_async_copy(hq_hbm.at[cols_v.at[sl]], GQ[b], SQ[b]).wait()
        for j in range(3):
            pltpu.make_async_copy(XH[j].at[rows_v.at[sl]], XR[b][j], SX[b]).wait()
            pltpu.make_async_copy(XH[j].at[cols_v.at[sl]], XC[b][j], SX[b]).wait()

    def wait_store(b):
        pltpu.make_async_copy(TV[b], t_hbm.at[pl.ds(ebase, CB)], ST[b]).wait()
        pltpu.make_async_copy(DV[b], d_hbm.at[pl.ds(ebase, CB)], SD[b]).wait()

    def compute(k, b):
        def dij_body(g, c2):
            sl = pl.ds(g * 16, 16)
            d0 = XR[b][0][sl] - XC[b][0][sl]
            d1 = XR[b][1][sl] - XC[b][1][sl]
            d2 = XR[b][2][sl] - XC[b][2][sl]
            DV[b][sl] = d0 * d0 + d1 * d1 + d2 * d2
            return c2

        lax.fori_loop(0, CB // 16, dij_body, 0, unroll=True)

        def edge_body(e, c2):
            for c in range(D // 16):
                sl = pl.ds(c * 16, 16)
                TV[b][e, sl] = GP[b][e, sl] + GQ[b][e, sl]
            return c2

        lax.fori_loop(0, CB, edge_body, 0, unroll=2)
        pltpu.async_copy(TV[b], t_hbm.at[pl.ds(ebase + k * CB, CB)], ST[b])
        pltpu.async_copy(DV[b], d_hbm.at[pl.ds(ebase + k * CB, CB)], SD[b])

    issue(0, 0)

    def pair_body(j, carry):
        for ph in range(2):
            k = 2 * j + ph
            b = ph

            @pl.when(k + 1 < nchunk)
            def _():
                issue(k + 1, 1 - ph)

            @pl.when(k < nchunk)
            def _():
                wait_gathers(b)

            @pl.when(jnp.logical_and(k >= 2, k < nchunk))
            def _():
                wait_store(b)

            @pl.when(k < nchunk)
            def _():
                compute(k, b)

        return carry

    lax.fori_loop(0, (nchunk + 2) // 2, pair_body, 0)
    wait_store(0)
    wait_store(1)


def _edge_gather(hp, hq, xx, xy, xz, row, col, e_off, e_cnt):
    epw = e_cnt // NW
    nchunk = epw // CB
    mesh = plsc.VectorSubcoreMesh(core_axis_name="c", subcore_axis_name="s")
    kern = functools.partial(
        pl.kernel,
        mesh=mesh,
        out_type=(
            jax.ShapeDtypeStruct((e_cnt, D), jnp.float32),
            jax.ShapeDtypeStruct((e_cnt,), jnp.float32),
        ),
        scratch_types=(
            [
                pltpu.VMEM((epw,), jnp.int32),
                pltpu.VMEM((epw,), jnp.int32),
            ]
            + [pltpu.VMEM((CB, D), jnp.float32)] * 6
            + [pltpu.VMEM((CB,), jnp.float32)] * 2
            + [pltpu.VMEM((CB,), jnp.float32)] * 12
            + [pltpu.SemaphoreType.DMA] * 10
        ),
    )(functools.partial(_edge_gather_body, e_off, epw, nchunk))
    return kern(hp, hq, xx, xy, xz, row, col)


def _mlp_body(t_ref, d_ref, w1c_ref, w2_ref, b2_ref, m_ref):
    a = _silu(t_ref[...] + d_ref[...] * w1c_ref[...])
    z = jnp.dot(a, w2_ref[...], preferred_element_type=jnp.float32) + b2_ref[...]
    m_ref[...] = _silu(z)


def _edge_mlp(t, dij, w1c, w2, b2):
    blk = 1280
    ecnt = t.shape[0]
    return pl.pallas_call(
        _mlp_body,
        grid=(ecnt // blk,),
        in_specs=[
            pl.BlockSpec((blk, D), lambda i: (i, 0)),
            pl.BlockSpec((blk, 1), lambda i: (i, 0)),
            pl.BlockSpec((1, D), lambda i: (0, 0)),
            pl.BlockSpec((D, D), lambda i: (0, 0)),
            pl.BlockSpec((1, D), lambda i: (0, 0)),
        ],
        out_specs=pl.BlockSpec((blk, D), lambda i: (i, 0)),
        out_shape=jax.ShapeDtypeStruct((ecnt, D), jnp.float32),
    )(t, dij, w1c, w2, b2)


WB = 624
WREM = N - NS * WB


def _scatter_body(ma_hbm, mb_hbm, row_hbm, zeros_hbm, out_hbm,
                  row0, row1, m0, m1, agg_sh, sf0, sf1):
    c = lax.axis_index("c")
    s = lax.axis_index("s")
    wid = s * NC + c
    ebase = wid * EPW
    RV = [row0, row1]
    MV = [m0, m1]
    SF = [sf0, sf1]

    def fetch(k, b):
        base = ebase + k * CB
        pltpu.async_copy(row_hbm.at[pl.ds(base, CB)], RV[b], SF[b])

        @pl.when(base < ESPLIT)
        def _():
            pltpu.async_copy(ma_hbm.at[pl.ds(base, CB)], MV[b], SF[b])

        @pl.when(base >= ESPLIT)
        def _():
            pltpu.async_copy(mb_hbm.at[pl.ds(base - ESPLIT, CB)], MV[b], SF[b])

    def wait_fetch(b):
        pltpu.make_async_copy(row_hbm.at[pl.ds(ebase, CB)], RV[b], SF[b]).wait()
        pltpu.make_async_copy(ma_hbm.at[pl.ds(ebase, CB)], MV[b], SF[b]).wait()

    fetch(0, 0)
    pltpu.sync_copy(zeros_hbm, agg_sh.at[pl.ds(s * WB, WB)])

    @pl.when(s == 0)
    def _():
        pltpu.sync_copy(zeros_hbm.at[pl.ds(0, WREM)], agg_sh.at[pl.ds(NS * WB, WREM)])

    plsc.subcore_barrier()

    def pair_body(j, carry):
        for ph in range(2):
            k = 2 * j + ph
            b = ph

            @pl.when(k + 1 < NCHUNK)
            def _():
                fetch(k + 1, 1 - ph)

            @pl.when(k < NCHUNK)
            def _():
                wait_fetch(b)
                pltpu.sync_copy(MV[b], agg_sh.at[RV[b]], add=True)

        return carry

    lax.fori_loop(0, (NCHUNK + 2) // 2, pair_body, 0)
    plsc.subcore_barrier()
    pltpu.sync_copy(agg_sh.at[pl.ds(s * WB, WB)], out_hbm.at[c, pl.ds(s * WB, WB)])

    @pl.when(s == 0)
    def _():
        pltpu.sync_copy(agg_sh.at[pl.ds(NS * WB, WREM)],
                        out_hbm.at[c, pl.ds(NS * WB, WREM)])


def _scatter_add(ma, mb, row, zeros):
    mesh = plsc.VectorSubcoreMesh(core_axis_name="c", subcore_axis_name="s")
    kern = functools.partial(
        pl.kernel,
        mesh=mesh,
        out_type=jax.ShapeDtypeStruct((NC, N, D), jnp.float32),
        scratch_types=[
            pltpu.VMEM((CB,), jnp.int32),
            pltpu.VMEM((CB,), jnp.int32),
            pltpu.VMEM((CB, D), jnp.float32),
            pltpu.VMEM((CB, D), jnp.float32),
            pltpu.VMEM_SHARED((N, D), jnp.float32),
            pltpu.SemaphoreType.DMA,
            pltpu.SemaphoreType.DMA,
        ],
    )(_scatter_body)
    return kern(ma, mb, row, zeros)


def _node_body(h_ref, p_ref, w3h_ref, w3a_ref, b3_ref, w4_ref, b4_ref,
               gamma_ref, beta_ref, o_ref):
    h = h_ref[...]
    p = p_ref[...]
    agg = (p[0] + p[1]) * (1.0 / K)
    z = (jnp.dot(h, w3h_ref[...], preferred_element_type=jnp.float32)
         + jnp.dot(agg, w3a_ref[...], preferred_element_type=jnp.float32)
         + b3_ref[...])
    u = jnp.dot(_silu(z), w4_ref[...], preferred_element_type=jnp.float32) + b4_ref[...]
    y = h + u
    mu = jnp.mean(y, axis=-1, keepdims=True)
    var = jnp.mean((y - mu) ** 2, axis=-1, keepdims=True)
    o_ref[...] = (y - mu) / jnp.sqrt(var + EPS) * gamma_ref[...] + beta_ref[...]


def _node_mlp(h, partials, w3h, w3a, b3, w4, b4, gamma, beta):
    blk = 2000
    return pl.pallas_call(
        _node_body,
        grid=(N // blk,),
        in_specs=[
            pl.BlockSpec((blk, D), lambda i: (i, 0)),
            pl.BlockSpec((NC, blk, D), lambda i: (0, i, 0)),
            pl.BlockSpec((D, D), lambda i: (0, 0)),
            pl.BlockSpec((D, D), lambda i: (0, 0)),
            pl.BlockSpec((1, D), lambda i: (0, 0)),
            pl.BlockSpec((D, D), lambda i: (0, 0)),
            pl.BlockSpec((1, D), lambda i: (0, 0)),
            pl.BlockSpec((1, D), lambda i: (0, 0)),
            pl.BlockSpec((1, D), lambda i: (0, 0)),
        ],
        out_specs=pl.BlockSpec((blk, D), lambda i: (i, 0)),
        out_shape=jax.ShapeDtypeStruct((N, D), jnp.float32),
    )(h, partials, w3h, w3a, b3, w4, b4, gamma, beta)


def kernel(h, x, edge_index, W1, b1, W2, b2, W3, b3, W4, b4, gamma, beta):
    row = edge_index[0].astype(jnp.int32)
    col = edge_index[1].astype(jnp.int32)
    xf = x.astype(jnp.float32)
    w1a = W1[:D]
    w1b = W1[D:2 * D]
    w1c = W1[2 * D]
    hp, hq = _node_pre(h, w1a, w1b, b1.reshape(1, D))
    w1c_b = w1c.reshape(1, D)
    b2_b = b2.reshape(1, D)
    t0, d0 = _edge_gather(hp, hq, xf[:, 0], xf[:, 1], xf[:, 2], row, col,
                          0, ESPLIT)
    t1, d1 = _edge_gather(hp, hq, xf[:, 0], xf[:, 1], xf[:, 2], row, col,
                          ESPLIT, E - ESPLIT)
    m0 = _edge_mlp(t0, d0.reshape(ESPLIT, 1), w1c_b, W2, b2_b)
    m1 = _edge_mlp(t1, d1.reshape(E - ESPLIT, 1), w1c_b, W2, b2_b)
    zeros = jnp.zeros((WB, D), jnp.float32)
    partials = _scatter_add(m0, m1, row, zeros)
    return _node_mlp(h, partials, W3[:D], W3[D:], b3.reshape(1, D), W4,
                     b4.reshape(1, D), gamma.reshape(1, D), beta.reshape(1, D))


# speedup vs baseline: 1.2759x; 1.1921x over previous
import functools

import jax
import jax.numpy as jnp
import numpy as np
from jax import lax
from jax.experimental import pallas as pl
from jax.experimental.pallas import tpu as pltpu
from jax.experimental.pallas import tpu_sc as plsc

N = 10000
E = 320000
D = 128
DW = D // 2
K = 32.0
EPS = 1e-5

NC = 2
NS = 16
NW = NC * NS
EPW = E // NW
CB = 80
NCHUNK = EPW // CB
ESPLIT = 163840
assert ESPLIT % (NW * CB) == 0 and (E - ESPLIT) % (NW * CB) == 0


def _silu(v):
    return v * jax.nn.sigmoid(v)


def _pre_body(h_ref, w1a_ref, w1b_ref, b1_ref, hp_ref, hq_ref):
    h = h_ref[...]
    hp_ref[...] = jnp.dot(h, w1a_ref[...], preferred_element_type=jnp.float32) + b1_ref[...]
    hq_ref[...] = jnp.dot(h, w1b_ref[...], preferred_element_type=jnp.float32)


def _node_pre(h, w1a, w1b, b1):
    blk = 2000
    return pl.pallas_call(
        _pre_body,
        grid=(N // blk,),
        in_specs=[
            pl.BlockSpec((blk, D), lambda i: (i, 0)),
            pl.BlockSpec((D, D), lambda i: (0, 0)),
            pl.BlockSpec((D, D), lambda i: (0, 0)),
            pl.BlockSpec((1, D), lambda i: (0, 0)),
        ],
        out_specs=[
            pl.BlockSpec((blk, D), lambda i: (i, 0)),
            pl.BlockSpec((blk, D), lambda i: (i, 0)),
        ],
        out_shape=[
            jax.ShapeDtypeStruct((N, D), jnp.float32),
            jax.ShapeDtypeStruct((N, D), jnp.float32),
        ],
    )(h, w1a, w1b, b1)


def _edge_gather_body(e_off, epw, nchunk,
                      hp_hbm, hq_hbm, xx_hbm, xy_hbm, xz_hbm, row_hbm, col_hbm,
                      t_hbm, d_hbm,
                      rows_v, cols_v,
                      gp0, gp1, gq0, gq1, t0, t1, dv0, dv1,
                      xr0a, xr1a, xr2a, xc0a, xc1a, xc2a,
                      xr0b, xr1b, xr2b, xc0b, xc1b, xc2b,
                      sp0, sp1, sq0, sq1, sx0, sx1, st0, st1, sd0, sd1):
    wid = lax.axis_index("s") * NC + lax.axis_index("c")
    ebase = wid * epw
    pltpu.sync_copy(row_hbm.at[pl.ds(e_off + ebase, epw)], rows_v)
    pltpu.sync_copy(col_hbm.at[pl.ds(e_off + ebase, epw)], cols_v)

    GP = [gp0, gp1]
    GQ = [gq0, gq1]
    TV = [t0, t1]
    DV = [dv0, dv1]
    XR = [[xr0a, xr1a, xr2a], [xr0b, xr1b, xr2b]]
    XC = [[xc0a, xc1a, xc2a], [xc0b, xc1b, xc2b]]
    SP = [sp0, sp1]
    SQ = [sq0, sq1]
    SX = [sx0, sx1]
    ST = [st0, st1]
    SD = [sd0, sd1]
    XH = [xx_hbm, xy_hbm, xz_hbm]

    def issue(k, b):
        sl = pl.ds(k * CB, CB)
        pltpu.async_copy(hp_hbm.at[rows_v.at[sl]], GP[b], SP[b])
        pltpu.async_copy(hq_hbm.at[cols_v.at[sl]], GQ[b], SQ[b])
        for j in range(3):
            pltpu.async_copy(XH[j].at[rows_v.at[sl]], XR[b][j], SX[b])
            pltpu.async_copy(XH[j].at[cols_v.at[sl]], XC[b][j], SX[b])

    def wait_gathers(b):
        sl = pl.ds(0, CB)
        pltpu.make_async_copy(hp_hbm.at[rows_v.at[sl]], GP[b], SP[b]).wait()
        pltpu.make_async_copy(hq_hbm.at[cols_v.at[sl]], GQ[b], SQ[b]).wait()
        for j in range(3):
            pltpu.make_async_copy(XH[j].at[rows_v.at[sl]], XR[b][j], SX[b]).wait()
            pltpu.make_async_copy(XH[j].at[cols_v.at[sl]], XC[b][j], SX[b]).wait()

    def wait_store(b):
        pltpu.make_async_copy(TV[b], t_hbm.at[pl.ds(ebase, CB)], ST[b]).wait()
        pltpu.make_async_copy(DV[b], d_hbm.at[pl.ds(ebase, CB)], SD[b]).wait()

    def compute(k, b):
        def dij_body(g, c2):
            sl = pl.ds(g * 16, 16)
            d0 = XR[b][0][sl] - XC[b][0][sl]
            d1 = XR[b][1][sl] - XC[b][1][sl]
            d2 = XR[b][2][sl] - XC[b][2][sl]
            DV[b][sl] = d0 * d0 + d1 * d1 + d2 * d2
            return c2

        lax.fori_loop(0, CB // 16, dij_body, 0, unroll=True)

        def edge_body(e, c2):
            for c in range(D // 16):
                sl = pl.ds(c * 16, 16)
                TV[b][e, sl] = GP[b][e, sl] + GQ[b][e, sl]
            return c2

        lax.fori_loop(0, CB, edge_body, 0, unroll=2)
        pltpu.async_copy(TV[b], t_hbm.at[pl.ds(ebase + k * CB, CB)], ST[b])
        pltpu.async_copy(DV[b], d_hbm.at[pl.ds(ebase + k * CB, CB)], SD[b])

    issue(0, 0)

    def pair_body(j, carry):
        for ph in range(2):
            k = 2 * j + ph
            b = ph

            @pl.when(k + 1 < nchunk)
            def _():
                issue(k + 1, 1 - ph)

            @pl.when(k < nchunk)
            def _():
                wait_gathers(b)

            @pl.when(jnp.logical_and(k >= 2, k < nchunk))
            def _():
                wait_store(b)

            @pl.when(k < nchunk)
            def _():
                compute(k, b)

        return carry

    lax.fori_loop(0, (nchunk + 2) // 2, pair_body, 0)
    wait_store(0)
    wait_store(1)


def _edge_gather(hp, hq, xx, xy, xz, row, col, e_off, e_cnt):
    epw = e_cnt // NW
    nchunk = epw // CB
    mesh = plsc.VectorSubcoreMesh(core_axis_name="c", subcore_axis_name="s")
    kern = functools.partial(
        pl.kernel,
        mesh=mesh,
        out_type=(
            jax.ShapeDtypeStruct((e_cnt, D), jnp.float32),
            jax.ShapeDtypeStruct((e_cnt,), jnp.float32),
        ),
        scratch_types=(
            [
                pltpu.VMEM((epw,), jnp.int32),
                pltpu.VMEM((epw,), jnp.int32),
            ]
            + [pltpu.VMEM((CB, D), jnp.float32)] * 6
            + [pltpu.VMEM((CB,), jnp.float32)] * 2
            + [pltpu.VMEM((CB,), jnp.float32)] * 12
            + [pltpu.SemaphoreType.DMA] * 10
        ),
    )(functools.partial(_edge_gather_body, e_off, epw, nchunk))
    return kern(hp, hq, xx, xy, xz, row, col)


def _mlp_body(t_ref, d_ref, w1c_ref, w2_ref, b2_ref, m_ref):
    a = _silu(t_ref[...] + d_ref[...] * w1c_ref[...])
    z = jnp.dot(a, w2_ref[...], preferred_element_type=jnp.float32) + b2_ref[...]
    m_ref[...] = _silu(z)


def _edge_mlp(t, dij, w1c, w2, b2):
    blk = 1280
    ecnt = t.shape[0]
    return pl.pallas_call(
        _mlp_body,
        grid=(ecnt // blk,),
        in_specs=[
            pl.BlockSpec((blk, D), lambda i: (i, 0)),
            pl.BlockSpec((blk, 1), lambda i: (i, 0)),
            pl.BlockSpec((1, D), lambda i: (0, 0)),
            pl.BlockSpec((D, D), lambda i: (0, 0)),
            pl.BlockSpec((1, D), lambda i: (0, 0)),
        ],
        out_specs=pl.BlockSpec((blk, D), lambda i: (i, 0)),
        out_shape=jax.ShapeDtypeStruct((ecnt, D), jnp.float32),
    )(t, dij, w1c, w2, b2)


WB = 624
WREM = N - NS * WB


def _scatter_body(e_off, epw, nchunk,
                  m_hbm, row_hbm, zeros_hbm, out_hbm,
                  row0, row1, m0, m1, agg_sh, sf0, sf1):
    c = lax.axis_index("c")
    s = lax.axis_index("s")
    wid = s * NC + c
    ebase = wid * epw
    RV = [row0, row1]
    MV = [m0, m1]
    SF = [sf0, sf1]

    def fetch(k, b):
        base = ebase + k * CB
        pltpu.async_copy(row_hbm.at[pl.ds(e_off + base, CB)], RV[b], SF[b])
        pltpu.async_copy(m_hbm.at[pl.ds(base, CB)], MV[b], SF[b])

    def wait_fetch(b):
        pltpu.make_async_copy(row_hbm.at[pl.ds(ebase, CB)], RV[b], SF[b]).wait()
        pltpu.make_async_copy(m_hbm.at[pl.ds(ebase, CB)], MV[b], SF[b]).wait()

    fetch(0, 0)
    pltpu.sync_copy(zeros_hbm, agg_sh.at[pl.ds(s * WB, WB)])

    @pl.when(s == 0)
    def _():
        pltpu.sync_copy(zeros_hbm.at[pl.ds(0, WREM)], agg_sh.at[pl.ds(NS * WB, WREM)])

    plsc.subcore_barrier()

    def pair_body(j, carry):
        for ph in range(2):
            k = 2 * j + ph
            b = ph

            @pl.when(k + 1 < nchunk)
            def _():
                fetch(k + 1, 1 - ph)

            @pl.when(k < nchunk)
            def _():
                wait_fetch(b)
                pltpu.sync_copy(MV[b], agg_sh.at[RV[b]], add=True)

        return carry

    lax.fori_loop(0, (nchunk + 2) // 2, pair_body, 0)
    plsc.subcore_barrier()
    pltpu.sync_copy(agg_sh.at[pl.ds(s * WB, WB)], out_hbm.at[c, pl.ds(s * WB, WB)])

    @pl.when(s == 0)
    def _():
        pltpu.sync_copy(agg_sh.at[pl.ds(NS * WB, WREM)],
                        out_hbm.at[c, pl.ds(NS * WB, WREM)])


def _scatter_add(m, row, zeros, e_off):
    e_cnt = m.shape[0]
    epw = e_cnt // NW
    nchunk = epw // CB
    mesh = plsc.VectorSubcoreMesh(core_axis_name="c", subcore_axis_name="s")
    kern = functools.partial(
        pl.kernel,
        mesh=mesh,
        out_type=jax.ShapeDtypeStruct((NC, N, D), jnp.float32),
        scratch_types=[
            pltpu.VMEM((CB,), jnp.int32),
            pltpu.VMEM((CB,), jnp.int32),
            pltpu.VMEM((CB, D), jnp.float32),
            pltpu.VMEM((CB, D), jnp.float32),
            pltpu.VMEM_SHARED((N, D), jnp.float32),
            pltpu.SemaphoreType.DMA,
            pltpu.SemaphoreType.DMA,
        ],
    )(functools.partial(_scatter_body, e_off, epw, nchunk))
    return kern(m, row, zeros)


def _node_body(h_ref, p_ref, q_ref, w3h_ref, w3a_ref, b3_ref, w4_ref, b4_ref,
               gamma_ref, beta_ref, o_ref):
    h = h_ref[...]
    p = p_ref[...]
    q = q_ref[...]
    agg = (p[0] + p[1] + q[0] + q[1]) * (1.0 / K)
    z = (jnp.dot(h, w3h_ref[...], preferred_element_type=jnp.float32)
         + jnp.dot(agg, w3a_ref[...], preferred_element_type=jnp.float32)
         + b3_ref[...])
    u = jnp.dot(_silu(z), w4_ref[...], preferred_element_type=jnp.float32) + b4_ref[...]
    y = h + u
    mu = jnp.mean(y, axis=-1, keepdims=True)
    var = jnp.mean((y - mu) ** 2, axis=-1, keepdims=True)
    o_ref[...] = (y - mu) / jnp.sqrt(var + EPS) * gamma_ref[...] + beta_ref[...]


def _node_mlp(h, p, q, w3h, w3a, b3, w4, b4, gamma, beta):
    blk = 2000
    return pl.pallas_call(
        _node_body,
        grid=(N // blk,),
        in_specs=[
            pl.BlockSpec((blk, D), lambda i: (i, 0)),
            pl.BlockSpec((NC, blk, D), lambda i: (0, i, 0)),
            pl.BlockSpec((NC, blk, D), lambda i: (0, i, 0)),
            pl.BlockSpec((D, D), lambda i: (0, 0)),
            pl.BlockSpec((D, D), lambda i: (0, 0)),
            pl.BlockSpec((1, D), lambda i: (0, 0)),
            pl.BlockSpec((D, D), lambda i: (0, 0)),
            pl.BlockSpec((1, D), lambda i: (0, 0)),
            pl.BlockSpec((1, D), lambda i: (0, 0)),
            pl.BlockSpec((1, D), lambda i: (0, 0)),
        ],
        out_specs=pl.BlockSpec((blk, D), lambda i: (i, 0)),
        out_shape=jax.ShapeDtypeStruct((N, D), jnp.float32),
    )(h, p, q, w3h, w3a, b3, w4, b4, gamma, beta)


def kernel(h, x, edge_index, W1, b1, W2, b2, W3, b3, W4, b4, gamma, beta):
    row = edge_index[0].astype(jnp.int32)
    col = edge_index[1].astype(jnp.int32)
    xf = x.astype(jnp.float32)
    w1a = W1[:D]
    w1b = W1[D:2 * D]
    w1c = W1[2 * D]
    hp, hq = _node_pre(h, w1a, w1b, b1.reshape(1, D))
    w1c_b = w1c.reshape(1, D)
    b2_b = b2.reshape(1, D)
    t0, d0 = _edge_gather(hp, hq, xf[:, 0], xf[:, 1], xf[:, 2], row, col,
                          0, ESPLIT)
    t1, d1 = _edge_gather(hp, hq, xf[:, 0], xf[:, 1], xf[:, 2], row, col,
                          ESPLIT, E - ESPLIT)
    m0 = _edge_mlp(t0, d0.reshape(ESPLIT, 1), w1c_b, W2, b2_b)
    m1 = _edge_mlp(t1, d1.reshape(E - ESPLIT, 1), w1c_b, W2, b2_b)
    zeros = jnp.zeros((WB, D), jnp.float32)
    p = _scatter_add(m0, row, zeros, 0)
    q = _scatter_add(m1, row, zeros, ESPLIT)
    return _node_mlp(h, p, q, W3[:D], W3[D:], b3.reshape(1, D), W4,
                     b4.reshape(1, D), gamma.reshape(1, D), beta.reshape(1, D))
